# baseline (speedup 1.0000x reference)
import functools

import jax
import jax.numpy as jnp
from jax import lax
from jax.experimental import pallas as pl
from jax.experimental.pallas import tpu as pltpu
from jax.experimental.pallas import tpu_sc as plsc

N = 10000
E = 320000
D = 128
C = 64

NC = 2
NS = 16
NW = NC * NS
EW = E // NW
K = 64
EWP = 10240
PAD = EWP - EW
NCH = EWP // K
NB = 8
NBLK = NCH // NB
ROWS0 = 624
ZR = 48
TAIL0 = NS * ROWS0
TAIL = N - TAIL0

_F32 = jnp.float32
_I32 = jnp.int32


def _sc_scatter_fn():
    mesh = plsc.VectorSubcoreMesh(
        core_axis_name="c", subcore_axis_name="s", num_cores=NC, num_subcores=NS
    )

    @functools.partial(
        pl.kernel,
        out_type=jax.ShapeDtypeStruct((NC, N, D), _F32),
        mesh=mesh,
        scratch_types=dict(
            idx_v=pltpu.VMEM((2, NB, 3, K), _I32),
            rows=pltpu.VMEM((2, K, D), _F32),
            acc=pltpu.VMEM_SHARED((N, D), _F32),
            ssem=pltpu.SemaphoreType.DMA,
        ),
    )
    def sc_scatter(x_hbm, idx_hbm, out_hbm, idx_v, rows, acc, ssem):
        c = lax.axis_index("c")
        s = lax.axis_index("s")
        wid = c * NS + s

        @pl.loop(0, ZR)
        def _(i):
            for j in range(D // 16):
                rows[0, i, pl.ds(j * 16, 16)] = jnp.zeros((16,), _F32)

        zsrc = rows.at[0, pl.ds(0, ZR)]
        row0 = s * ROWS0

        @pl.loop(0, ROWS0 // ZR)
        def _(i):
            pltpu.sync_copy(zsrc, acc.at[pl.ds(row0 + i * ZR, ZR)])

        @pl.when(s == NS - 1)
        def _():
            pltpu.sync_copy(rows.at[0, pl.ds(0, TAIL)],
                            acc.at[pl.ds(TAIL0, TAIL)])

        plsc.subcore_barrier()

        @pl.loop(0, NCH + 1)
        def _(gg):
            b = gg % 2
            sl = (gg // NB) % 2

            @pl.when(gg < NCH)
            def _():
                @pl.when(gg % NB == 0)
                def _():
                    pltpu.sync_copy(idx_hbm.at[wid, gg // NB], idx_v.at[sl])

                pltpu.sync_copy(x_hbm.at[idx_v.at[sl, gg % NB, 0]],
                                rows.at[b])

                @pl.loop(0, K // 16)
                def _(tt):
                    wvec = lax.bitcast_convert_type(
                        idx_v[sl, gg % NB, 2, pl.ds(tt * 16, 16)], _F32)
                    for l in range(16):
                        wb = jnp.full((16,), wvec[l], dtype=_F32)
                        row = tt * 16 + l
                        for j in range(D // 16):
                            slc = pl.ds(j * 16, 16)
                            rows[b, row, slc] = rows[b, row, slc] * wb

            @pl.when(gg >= 1)
            def _():
                pltpu.make_async_copy(
                    rows.at[1 - b], acc.at[idx_v.at[0, 0, 1]], ssem).wait()

            @pl.when(gg < NCH)
            def _():
                pltpu.async_copy(rows.at[b],
                                 acc.at[idx_v.at[sl, gg % NB, 1]],
                                 ssem, add=True)

        plsc.subcore_barrier()

        obuf = rows.at[0, pl.ds(0, ZR)]

        @pl.loop(0, ROWS0 // ZR)
        def _(i):
            r0 = row0 + i * ZR
            pltpu.sync_copy(acc.at[pl.ds(r0, ZR)], obuf)
            pltpu.sync_copy(obuf, out_hbm.at[c, pl.ds(r0, ZR)])

        @pl.when(s == NS - 1)
        def _():
            tbuf = rows.at[0, pl.ds(0, TAIL)]
            pltpu.sync_copy(acc.at[pl.ds(TAIL0, TAIL)], tbuf)
            pltpu.sync_copy(tbuf, out_hbm.at[c, pl.ds(TAIL0, TAIL)])

    return sc_scatter


_SC_SCATTER = _sc_scatter_fn()

BT = 2000


def _dotT(a, w):
    return lax.dot_general(a, w, (((1,), (1,)), ((), ())),
                           preferred_element_type=_F32)


def _layer_body(p_ref, x_ref, wrel_ref, b_ref, wroot_ref, o_ref):
    agg = p_ref[0] + p_ref[1]
    t = _dotT(agg, wrel_ref[...]) + _dotT(x_ref[...], wroot_ref[...]) + b_ref[...]
    o_ref[...] = jnp.maximum(t, 0.0)


def _tc_layer(p, x, w_rel, b_rel, w_root):
    return pl.pallas_call(
        _layer_body,
        grid=(N // BT,),
        in_specs=[
            pl.BlockSpec((NC, BT, D), lambda i: (0, i, 0)),
            pl.BlockSpec((BT, D), lambda i: (i, 0)),
            pl.BlockSpec((D, D), lambda i: (0, 0)),
            pl.BlockSpec((1, D), lambda i: (0, 0)),
            pl.BlockSpec((D, D), lambda i: (0, 0)),
        ],
        out_specs=pl.BlockSpec((BT, D), lambda i: (i, 0)),
        out_shape=jax.ShapeDtypeStruct((N, D), _F32),
    )(p, x, w_rel, b_rel.reshape(1, D), w_root)


def _final_body(p_ref, h_ref, wrel_ref, b_ref, wroot_ref, wfc_ref, bfc_ref, o_ref):
    agg = p_ref[0] + p_ref[1]
    h2 = jnp.maximum(
        _dotT(agg, wrel_ref[...]) + _dotT(h_ref[...], wroot_ref[...]) + b_ref[...],
        0.0,
    )
    o_ref[...] = _dotT(h2, wfc_ref[...]) + bfc_ref[...]


def _tc_final(p, h, w_rel, b_rel, w_root, wfc, bfc):
    return pl.pallas_call(
        _final_body,
        grid=(N // BT,),
        in_specs=[
            pl.BlockSpec((NC, BT, D), lambda i: (0, i, 0)),
            pl.BlockSpec((BT, D), lambda i: (i, 0)),
            pl.BlockSpec((D, D), lambda i: (0, 0)),
            pl.BlockSpec((1, D), lambda i: (0, 0)),
            pl.BlockSpec((D, D), lambda i: (0, 0)),
            pl.BlockSpec((C, D), lambda i: (0, 0)),
            pl.BlockSpec((1, C), lambda i: (0, 0)),
        ],
        out_specs=pl.BlockSpec((BT, C), lambda i: (i, 0)),
        out_shape=jax.ShapeDtypeStruct((N, C), _F32),
    )(p, h, w_rel, b_rel.reshape(1, D), w_root, wfc, bfc.reshape(1, C))


def kernel(x, edge_index, edge_attr, W1_rel, b1_rel, W1_root,
           W2_rel, b2_rel, W2_root, Wfc, bfc):
    pad = ((0, 0), (0, PAD))
    src = jnp.pad(edge_index[0].reshape(NW, EW), pad)
    dst = jnp.pad(edge_index[1].reshape(NW, EW), pad)
    wbits = jnp.pad(
        lax.bitcast_convert_type(edge_attr, _I32).reshape(NW, EW), pad)
    idx = jnp.stack(
        [a.reshape(NW, NCH, K) for a in (src, dst, wbits)], axis=2
    ).reshape(NW, NBLK, NB, 3, K)

    p1 = _SC_SCATTER(x, idx)
    h1 = _tc_layer(p1, x, W1_rel, b1_rel, W1_root)
    p2 = _SC_SCATTER(h1, idx)
    return _tc_final(p2, h1, W2_rel, b2_rel, W2_root, Wfc, bfc)


# speedup vs baseline: 2.9270x; 2.9270x over previous
import functools

import jax
import jax.numpy as jnp
from jax import lax
from jax.experimental import pallas as pl
from jax.experimental.pallas import tpu as pltpu
from jax.experimental.pallas import tpu_sc as plsc

N = 10000
E = 320000
D = 128
C = 64

NC = 2
NS = 16
NW = NC * NS
EW = E // NW
K = 80
EWP = 10000
PAD = EWP - EW
NCH = EWP // K
NB = 25
NBLK = NCH // NB
ROWS0 = 624
ZR = 48
TAIL0 = NS * ROWS0
TAIL = N - TAIL0

_F32 = jnp.float32
_I32 = jnp.int32


def _sc_scatter_fn():
    mesh = plsc.VectorSubcoreMesh(
        core_axis_name="c", subcore_axis_name="s", num_cores=NC, num_subcores=NS
    )

    @functools.partial(
        pl.kernel,
        out_type=jax.ShapeDtypeStruct((NC, N, D), _F32),
        mesh=mesh,
        scratch_types=dict(
            idx_v=pltpu.VMEM((NB, 3, K), _I32),
            rows=pltpu.VMEM((K, D), _F32),
            acc=pltpu.VMEM_SHARED((N, D), _F32),
        ),
    )
    def sc_scatter(x_hbm, idx_hbm, out_hbm, idx_v, rows, acc):
        c = lax.axis_index("c")
        s = lax.axis_index("s")
        wid = c * NS + s

        @pl.loop(0, ZR)
        def _(i):
            for j in range(D // 16):
                rows[i, pl.ds(j * 16, 16)] = jnp.zeros((16,), _F32)

        zsrc = rows.at[pl.ds(0, ZR)]
        row0 = s * ROWS0

        @pl.loop(0, ROWS0 // ZR)
        def _(i):
            pltpu.sync_copy(zsrc, acc.at[pl.ds(row0 + i * ZR, ZR)])

        @pl.when(s == NS - 1)
        def _():
            pltpu.sync_copy(rows.at[pl.ds(0, TAIL)], acc.at[pl.ds(TAIL0, TAIL)])

        plsc.subcore_barrier()

        @pl.loop(0, NBLK)
        def _(blk):
            pltpu.sync_copy(idx_hbm.at[wid, blk], idx_v)

            @pl.loop(0, NB)
            def _(gg):
                pltpu.sync_copy(x_hbm.at[idx_v.at[gg, 0]], rows)

                @pl.loop(0, K // 16)
                def _(tt):
                    wvec = lax.bitcast_convert_type(
                        idx_v[gg, 2, pl.ds(tt * 16, 16)], _F32)
                    for l in range(16):
                        wb = jnp.full((16,), wvec[l], dtype=_F32)
                        row = tt * 16 + l
                        for j in range(D // 16):
                            slc = pl.ds(j * 16, 16)
                            rows[row, slc] = rows[row, slc] * wb

                pltpu.sync_copy(rows, acc.at[idx_v.at[gg, 1]], add=True)

        plsc.subcore_barrier()

        obuf = rows.at[pl.ds(0, ZR)]

        @pl.loop(0, ROWS0 // ZR)
        def _(i):
            r0 = row0 + i * ZR
            pltpu.sync_copy(acc.at[pl.ds(r0, ZR)], obuf)
            pltpu.sync_copy(obuf, out_hbm.at[c, pl.ds(r0, ZR)])

        @pl.when(s == NS - 1)
        def _():
            tbuf = rows.at[pl.ds(0, TAIL)]
            pltpu.sync_copy(acc.at[pl.ds(TAIL0, TAIL)], tbuf)
            pltpu.sync_copy(tbuf, out_hbm.at[c, pl.ds(TAIL0, TAIL)])

    return sc_scatter


_SC_SCATTER = _sc_scatter_fn()

BT = 2000


def _dotT(a, w):
    return lax.dot_general(a, w, (((1,), (1,)), ((), ())),
                           preferred_element_type=_F32)


def _layer_body(p_ref, x_ref, wrel_ref, b_ref, wroot_ref, o_ref):
    agg = p_ref[0] + p_ref[1]
    t = _dotT(agg, wrel_ref[...]) + _dotT(x_ref[...], wroot_ref[...]) + b_ref[...]
    o_ref[...] = jnp.maximum(t, 0.0)


def _tc_layer(p, x, w_rel, b_rel, w_root):
    return pl.pallas_call(
        _layer_body,
        grid=(N // BT,),
        in_specs=[
            pl.BlockSpec((NC, BT, D), lambda i: (0, i, 0)),
            pl.BlockSpec((BT, D), lambda i: (i, 0)),
            pl.BlockSpec((D, D), lambda i: (0, 0)),
            pl.BlockSpec((1, D), lambda i: (0, 0)),
            pl.BlockSpec((D, D), lambda i: (0, 0)),
        ],
        out_specs=pl.BlockSpec((BT, D), lambda i: (i, 0)),
        out_shape=jax.ShapeDtypeStruct((N, D), _F32),
    )(p, x, w_rel, b_rel.reshape(1, D), w_root)


def _final_body(p_ref, h_ref, wrel_ref, b_ref, wroot_ref, wfc_ref, bfc_ref, o_ref):
    agg = p_ref[0] + p_ref[1]
    h2 = jnp.maximum(
        _dotT(agg, wrel_ref[...]) + _dotT(h_ref[...], wroot_ref[...]) + b_ref[...],
        0.0,
    )
    o_ref[...] = _dotT(h2, wfc_ref[...]) + bfc_ref[...]


def _tc_final(p, h, w_rel, b_rel, w_root, wfc, bfc):
    return pl.pallas_call(
        _final_body,
        grid=(N // BT,),
        in_specs=[
            pl.BlockSpec((NC, BT, D), lambda i: (0, i, 0)),
            pl.BlockSpec((BT, D), lambda i: (i, 0)),
            pl.BlockSpec((D, D), lambda i: (0, 0)),
            pl.BlockSpec((1, D), lambda i: (0, 0)),
            pl.BlockSpec((D, D), lambda i: (0, 0)),
            pl.BlockSpec((C, D), lambda i: (0, 0)),
            pl.BlockSpec((1, C), lambda i: (0, 0)),
        ],
        out_specs=pl.BlockSpec((BT, C), lambda i: (i, 0)),
        out_shape=jax.ShapeDtypeStruct((N, C), _F32),
    )(p, h, w_rel, b_rel.reshape(1, D), w_root, wfc, bfc.reshape(1, C))


def kernel(x, edge_index, edge_attr, W1_rel, b1_rel, W1_root,
           W2_rel, b2_rel, W2_root, Wfc, bfc):
    pad = ((0, 0), (0, PAD))
    src = jnp.pad(edge_index[0].reshape(NW, EW), pad)
    dst = jnp.pad(edge_index[1].reshape(NW, EW), pad)
    wbits = jnp.pad(
        lax.bitcast_convert_type(edge_attr, _I32).reshape(NW, EW), pad)
    idx = jnp.stack(
        [a.reshape(NW, NCH, K) for a in (src, dst, wbits)], axis=2
    ).reshape(NW, NBLK, NB, 3, K)

    p1 = _SC_SCATTER(x, idx)
    h1 = _tc_layer(p1, x, W1_rel, b1_rel, W1_root)
    p2 = _SC_SCATTER(h1, idx)
    return _tc_final(p2, h1, W2_rel, b2_rel, W2_root, Wfc, bfc)
